# baseline (speedup 1.0000x reference)
import jax
import jax.numpy as jnp
from jax.experimental import pallas as pl


_COO = ((0, 0, 1.0), (1, 1, 2.0), (2, 2, 3.0))
_BC = 16384


def _add_body(x_ref, o_ref):
    j = jax.lax.broadcasted_iota(jnp.int32, x_ref.shape, 0)
    k = jax.lax.broadcasted_iota(jnp.int32, x_ref.shape, 1)
    c = jnp.zeros(x_ref.shape, jnp.float32)
    for r, cc, val in _COO:
        c = c + jnp.where((j == r) & (k == cc), jnp.float32(val), jnp.float32(0.0))
    o_ref[...] = x_ref[...] + c


def kernel(x):
    n = x.shape[0]
    xt = x.transpose(1, 2, 0)
    bc = min(_BC, n)
    out_t = pl.pallas_call(
        _add_body,
        grid=(n // bc,),
        in_specs=[pl.BlockSpec((4, 4, bc), lambda i: (0, 0, i))],
        out_specs=pl.BlockSpec((4, 4, bc), lambda i: (0, 0, i)),
        out_shape=jax.ShapeDtypeStruct((4, 4, n), x.dtype),
    )(xt)
    return out_t.transpose(2, 0, 1)


# speedup vs baseline: 1.5744x; 1.5744x over previous
import jax
import jax.numpy as jnp
from jax.experimental import pallas as pl


_COO = ((0, 0, 1.0), (1, 1, 2.0), (2, 2, 3.0))
_BC = 131072


def _add_body(x_ref, o_ref):
    j = jax.lax.broadcasted_iota(jnp.int32, x_ref.shape, 0)
    k = jax.lax.broadcasted_iota(jnp.int32, x_ref.shape, 1)
    c = jnp.zeros(x_ref.shape, jnp.float32)
    for r, cc, val in _COO:
        c = c + jnp.where((j == r) & (k == cc), jnp.float32(val), jnp.float32(0.0))
    o_ref[...] = x_ref[...] + c


def kernel(x):
    n = x.shape[0]
    xt = x.transpose(1, 2, 0)
    bc = min(_BC, n)
    out_t = pl.pallas_call(
        _add_body,
        grid=(n // bc,),
        in_specs=[pl.BlockSpec((4, 4, bc), lambda i: (0, 0, i))],
        out_specs=pl.BlockSpec((4, 4, bc), lambda i: (0, 0, i)),
        out_shape=jax.ShapeDtypeStruct((4, 4, n), x.dtype),
    )(xt)
    return out_t.transpose(2, 0, 1)
